# baseline (speedup 1.0000x reference)
import jax
import jax.numpy as jnp
from jax import lax
from jax.experimental import pallas as pl
from jax.experimental.pallas import tpu as pltpu
from jax.experimental.pallas import tpu_sc as plsc

_B, _H, _W = 8, 512, 512
_RBLK = 256


def _geo_kernel(score_ref, geo_ref, pred_geo_ref, edge_ref, out_ref, acc_ref):
    b = pl.program_id(0)
    i = pl.program_id(1)
    ni = pl.num_programs(1)

    @pl.when(i == 0)
    def _init_batch():
        acc_ref[0] = 0.0
        acc_ref[1] = 0.0

    @pl.when((b == 0) & (i == 0))
    def _init_total():
        acc_ref[2] = 0.0

    score = score_ref[0]
    edge = edge_ref[0]
    x = geo_ref[0] - pred_geo_ref[0]
    y = jnp.clip(x, -1.0, 1.0)
    sl1 = y * (x - 0.5 * y)
    chsum = jnp.sum(sl1, axis=0)
    mask = (score != 0.0).astype(jnp.float32)
    w = mask * (0.125 / edge)

    acc_ref[0] += jnp.sum(mask)
    acc_ref[1] += jnp.sum(chsum * w)

    @pl.when(i == ni - 1)
    def _finish_batch():
        acc_ref[2] += acc_ref[1] / jnp.maximum(acc_ref[0], 1.0)

        @pl.when(b == _B - 1)
        def _finalize():
            out_ref[0] = acc_ref[2] / float(_B)


def _geo_loss_mean(score, geo, pred_geo, edge):
    grid = (_B, _H // _RBLK)
    out = pl.pallas_call(
        _geo_kernel,
        grid=grid,
        in_specs=[
            pl.BlockSpec((1, _RBLK, _W), lambda b, i: (b, i, 0)),
            pl.BlockSpec((1, 8, _RBLK, _W), lambda b, i: (b, 0, i, 0)),
            pl.BlockSpec((1, 8, _RBLK, _W), lambda b, i: (b, 0, i, 0)),
            pl.BlockSpec((1, _RBLK, _W), lambda b, i: (b, i, 0)),
        ],
        out_specs=pl.BlockSpec(memory_space=pltpu.SMEM),
        out_shape=jax.ShapeDtypeStruct((1,), jnp.float32),
        scratch_shapes=[pltpu.SMEM((3,), jnp.float32)],
    )(score, geo, pred_geo, edge)
    return out[0]


_NC, _NS, _L = 2, 16, 16
_NW = _NC * _NS
_PER_W = (_B * _H * _W) // _NW
_CHUNK = 16384
_NCHUNK = _PER_W // _CHUNK


def _dice_partials(score, pred_score):
    s_flat = score.reshape(-1)
    p_flat = pred_score.reshape(-1)
    mesh = plsc.VectorSubcoreMesh(core_axis_name="c", subcore_axis_name="s")

    def run(s_hbm_arr, p_hbm_arr):
        @pl.kernel(
            out_type=jax.ShapeDtypeStruct((_NW, 3 * _L), jnp.float32),
            mesh=mesh,
            scratch_types=[
                pltpu.VMEM((_CHUNK,), jnp.float32),
                pltpu.VMEM((_CHUNK,), jnp.float32),
                pltpu.VMEM((3 * _L,), jnp.float32),
            ],
        )
        def k(s_hbm, p_hbm, out_hbm, sbuf, pbuf, obuf):
            wid = lax.axis_index("s") * _NC + lax.axis_index("c")
            base = wid * _PER_W

            def chunk_body(c, accs):
                acc_a, acc_b, acc_c = accs
                off = base + c * _CHUNK
                pltpu.sync_copy(s_hbm.at[pl.ds(off, _CHUNK)], sbuf)
                pltpu.sync_copy(p_hbm.at[pl.ds(off, _CHUNK)], pbuf)

                def vec_body(i, accs2):
                    a2, b2, c2 = accs2
                    s = sbuf[pl.ds(i * _L, _L)]
                    p = pbuf[pl.ds(i * _L, _L)]
                    return (a2 + s * p, b2 + s, c2 + p)

                return lax.fori_loop(0, _CHUNK // _L, vec_body,
                                     (acc_a, acc_b, acc_c))

            zero = jnp.zeros((_L,), jnp.float32)
            acc_a, acc_b, acc_c = lax.fori_loop(
                0, _NCHUNK, chunk_body, (zero, zero, zero))
            obuf[pl.ds(0, _L)] = acc_a
            obuf[pl.ds(_L, _L)] = acc_b
            obuf[pl.ds(2 * _L, _L)] = acc_c
            pltpu.sync_copy(obuf, out_hbm.at[wid])

        return k(s_hbm_arr, p_hbm_arr)

    return run(s_flat, p_flat)


def kernel(score, pred_score, geo, pred_geo, edge):
    partials = _dice_partials(score, pred_score)
    geo_mean = _geo_loss_mean(score, geo, pred_geo, edge)
    per_batch = partials.reshape(_B, _NW // _B, 3, _L).sum(axis=(1, 3))
    a, bs, c = per_batch[:, 0], per_batch[:, 1], per_batch[:, 2]
    dice = 1.0 - 2.0 * a / (bs + c)
    return jnp.mean(dice) + geo_mean


# speedup vs baseline: 1.2959x; 1.2959x over previous
import jax
import jax.numpy as jnp
from jax import lax
from jax.experimental import pallas as pl
from jax.experimental.pallas import tpu as pltpu
from jax.experimental.pallas import tpu_sc as plsc

_B, _H, _W = 8, 512, 512
_RBLK = 256


def _geo_kernel(score_ref, geo_ref, pred_geo_ref, edge_ref, out_ref, acc_ref):
    b = pl.program_id(0)
    i = pl.program_id(1)
    ni = pl.num_programs(1)

    @pl.when(i == 0)
    def _init_batch():
        acc_ref[0] = 0.0
        acc_ref[1] = 0.0

    @pl.when((b == 0) & (i == 0))
    def _init_total():
        acc_ref[2] = 0.0

    score = score_ref[0]
    edge = edge_ref[0]
    x = geo_ref[0] - pred_geo_ref[0]
    y = jnp.clip(x, -1.0, 1.0)
    sl1 = y * (x - 0.5 * y)
    chsum = jnp.sum(sl1, axis=0)
    mask = (score != 0.0).astype(jnp.float32)
    w = mask * (0.125 / edge)

    acc_ref[0] += jnp.sum(mask)
    acc_ref[1] += jnp.sum(chsum * w)

    @pl.when(i == ni - 1)
    def _finish_batch():
        acc_ref[2] += acc_ref[1] / jnp.maximum(acc_ref[0], 1.0)

        @pl.when(b == _B - 1)
        def _finalize():
            out_ref[0] = acc_ref[2] / float(_B)


def _geo_loss_mean(score, geo, pred_geo, edge):
    grid = (_B, _H // _RBLK)
    out = pl.pallas_call(
        _geo_kernel,
        grid=grid,
        in_specs=[
            pl.BlockSpec((1, _RBLK, _W), lambda b, i: (b, i, 0)),
            pl.BlockSpec((1, 8, _RBLK, _W), lambda b, i: (b, 0, i, 0)),
            pl.BlockSpec((1, 8, _RBLK, _W), lambda b, i: (b, 0, i, 0)),
            pl.BlockSpec((1, _RBLK, _W), lambda b, i: (b, i, 0)),
        ],
        out_specs=pl.BlockSpec(memory_space=pltpu.SMEM),
        out_shape=jax.ShapeDtypeStruct((1,), jnp.float32),
        scratch_shapes=[pltpu.SMEM((3,), jnp.float32)],
    )(score, geo, pred_geo, edge)
    return out[0]


_NC, _NS, _L = 2, 16, 16
_NW = _NC * _NS
_ROWS = _B * _H
_RPW = _ROWS // _NW
_CROWS = 32
_NCHUNK = _RPW // _CROWS


def _dice_partials(score, pred_score):
    s2d = score.reshape(_ROWS, _W)
    p2d = pred_score.reshape(_ROWS, _W)
    mesh = plsc.VectorSubcoreMesh(core_axis_name="c", subcore_axis_name="s")

    def run(s_hbm_arr, p_hbm_arr):
        @pl.kernel(
            out_type=jax.ShapeDtypeStruct((_NW, 3 * _L), jnp.float32),
            mesh=mesh,
            scratch_types=[
                pltpu.VMEM((_CROWS, _W), jnp.float32),
                pltpu.VMEM((_CROWS, _W), jnp.float32),
                pltpu.VMEM((3 * _L,), jnp.float32),
            ],
        )
        def k(s_hbm, p_hbm, out_hbm, sbuf, pbuf, obuf):
            wid = lax.axis_index("s") * _NC + lax.axis_index("c")
            base = wid * _RPW

            def chunk_body(c, accs):
                row0 = base + c * _CROWS
                pltpu.sync_copy(s_hbm.at[pl.ds(row0, _CROWS), :], sbuf)
                pltpu.sync_copy(p_hbm.at[pl.ds(row0, _CROWS), :], pbuf)

                def row_body(r, accs_r):
                    def vec_body(i, accs2):
                        a2, b2, c2 = accs2
                        s = sbuf[r, pl.ds(i * _L, _L)]
                        p = pbuf[r, pl.ds(i * _L, _L)]
                        return (a2 + s * p, b2 + s, c2 + p)

                    return lax.fori_loop(0, _W // _L, vec_body, accs_r)

                return lax.fori_loop(0, _CROWS, row_body, accs)

            zero = jnp.zeros((_L,), jnp.float32)
            acc_a, acc_b, acc_c = lax.fori_loop(
                0, _NCHUNK, chunk_body, (zero, zero, zero))
            obuf[pl.ds(0, _L)] = acc_a
            obuf[pl.ds(_L, _L)] = acc_b
            obuf[pl.ds(2 * _L, _L)] = acc_c
            pltpu.sync_copy(obuf, out_hbm.at[wid])

        return k(s_hbm_arr, p_hbm_arr)

    return run(s2d, p2d)


def kernel(score, pred_score, geo, pred_geo, edge):
    partials = _dice_partials(score, pred_score)
    geo_mean = _geo_loss_mean(score, geo, pred_geo, edge)
    per_batch = partials.reshape(_B, _NW // _B, 3, _L).sum(axis=(1, 3))
    a, bs, c = per_batch[:, 0], per_batch[:, 1], per_batch[:, 2]
    dice = 1.0 - 2.0 * a / (bs + c)
    return jnp.mean(dice) + geo_mean
